# baseline (speedup 1.0000x reference)
import jax
import jax.numpy as jnp
from jax import lax
from jax.experimental import pallas as pl
from jax.experimental.pallas import tpu as pltpu
from jax.experimental.pallas import tpu_sc as plsc

N = 10000
E = 320000
D = 128
H = 8
DH = D // H

NC = 2
NS = 16
NW = NC * NS
C = 48
NG = C // 16
NCHUNK = 210
NDBL = NCHUNK // 2
EPT = NCHUNK * C
EPAD = NW * EPT + C
NP = N + 16
AW = 10032
NZW = AW // C
AD = 1280
BN = 1000


_DNUMS = lax.GatherDimensionNumbers(
    offset_dims=(), collapsed_slice_dims=(0,), start_index_map=(0,))


def _splat_idx(vec, idx):
    return lax.gather(vec, idx, _DNUMS, (1,),
                      mode=lax.GatherScatterMode.PROMISE_IN_BOUNDS)


def _ln(x, g, b):
    m = jnp.mean(x, axis=-1, keepdims=True)
    v = jnp.mean((x - m) * (x - m), axis=-1, keepdims=True)
    return (x - m) / jnp.sqrt(v + 1e-5) * g + b


def _proj_body(xs_ref, xt_ref, wq, bq, wk, bk, wv, bv, g1, b1n,
               q_out, k_out, v_out):
    xs = _ln(xs_ref[...], g1[...], b1n[...])
    mask = (jnp.sum(xs, axis=-1, keepdims=True) != 0).astype(jnp.float32)
    q_out[...] = jnp.dot(xt_ref[...], wq[...],
                         preferred_element_type=jnp.float32) + bq[...]
    k_out[...] = (jnp.dot(xs, wk[...],
                          preferred_element_type=jnp.float32) + bk[...]) * 0.25
    v_out[...] = (jnp.dot(xs, wv[...],
                          preferred_element_type=jnp.float32) + bv[...]) * mask


def _stage1(x_src, x_tgt, Wq, bq, Wk, bk, Wv, bv, g1, b1n):
    mat = pl.BlockSpec((D, D), lambda i: (0, 0))
    vec = pl.BlockSpec((D,), lambda i: (0,))
    blk = pl.BlockSpec((BN, D), lambda i: (i, 0))
    return pl.pallas_call(
        _proj_body,
        grid=(N // BN,),
        in_specs=[blk, blk, mat, vec, mat, vec, mat, vec, vec, vec],
        out_specs=[blk, blk, blk],
        out_shape=[jax.ShapeDtypeStruct((N, D), jnp.float32)] * 3,
    )(x_src, x_tgt, Wq, bq, Wk, bk, Wv, bv, g1, b1n)


def _edge_body(q_hbm, k_hbm, v_hbm, src_hbm, dst_hbm, outwv_hbm, outd_hbm,
               sidx0, sidx1, didx0, didx1, didx2, qb, kb, vb0, vb1, wbuf,
               msgd, accwv, accd,
               sem_q, sem_k, sem_i1, sem_i2, sem_v0, sem_v1, sem_s0, sem_s1):
    cid = lax.axis_index("c")
    sid = lax.axis_index("s")
    zero16 = jnp.zeros((16,), jnp.float32)
    lane = lax.iota(jnp.int32, 16)
    lane15 = lane == 15
    mask8 = (lane < H).astype(jnp.float32)
    hvecs = [jnp.full((16,), h, jnp.int32) for h in range(H)]
    hsplat = [jnp.full((16, 1), h, jnp.int32) for h in range(H)]

    sidx = (sidx0, sidx1)
    didx = (didx0, didx1)
    vb = (vb0, vb1)
    sem_v = (sem_v0, sem_v1)
    sem_s = (sem_s0, sem_s1)

    def _zrow(i, _):
        for j in range(D // 16):
            vb0[i, pl.ds(j * 16, 16)] = zero16
        wbuf[i, pl.ds(0, 16)] = zero16
        return 0
    lax.fori_loop(0, C, _zrow, 0)

    def _zrow2(i, _):
        for j in range(D // 16):
            msgd[i, pl.ds(j * 16, 16)] = zero16
        return 0
    lax.fori_loop(0, C, _zrow2, 0)

    for t in range(14):
        m = sid * 14 + t

        @pl.when(m < NZW)
        def _():
            pltpu.sync_copy(vb0, accwv.at[pl.ds(m * C, C)])

    pltpu.sync_copy(vb0, accd.at[pl.ds(sid * 80, C)])
    pltpu.sync_copy(vb0.at[pl.ds(0, 32)], accd.at[pl.ds(sid * 80 + C, 32)])
    plsc.subcore_barrier()

    wid = sid * NC + cid
    base = wid * EPT

    pltpu.sync_copy(src_hbm.at[pl.ds(base, C)], sidx0)
    pltpu.sync_copy(dst_hbm.at[pl.ds(base, C)], didx0)
    pltpu.async_copy(q_hbm.at[didx0], qb, sem_q)
    pltpu.async_copy(k_hbm.at[sidx0], kb, sem_k)
    pltpu.async_copy(v_hbm.at[sidx0], vb0, sem_v0)

    def _half_chunk(j, p):
        sx, dx, vbp = sidx[p], didx[p], vb[p]
        sxn, dxn, vbn = sidx[1 - p], didx[1 - p], vb[1 - p]

        pltpu.make_async_copy(q_hbm.at[dx], qb, sem_q).wait()
        pltpu.make_async_copy(k_hbm.at[sx], kb, sem_k).wait()

        def _alpha(i, _):
            e0 = 2 * i
            e1 = 2 * i + 1
            ei0 = lane * 0 + e0
            ei1 = lane * 0 + e1
            qs0 = [qb[e0, pl.ds(h * 16, 16)] for h in range(H)]
            ks0 = [kb[e0, pl.ds(h * 16, 16)] for h in range(H)]
            css0 = [plsc.cumsum(qs0[h] * ks0[h]) for h in range(H)]
            qs1 = [qb[e1, pl.ds(h * 16, 16)] for h in range(H)]
            ks1 = [kb[e1, pl.ds(h * 16, 16)] for h in range(H)]
            for h in range(H):
                plsc.store_scatter(wbuf, [ei0, hvecs[h]], css0[h],
                                   mask=lane15)
            css1 = [plsc.cumsum(qs1[h] * ks1[h]) for h in range(H)]
            wv0 = jnp.exp(wbuf[e0, pl.ds(0, 16)])
            wbuf[e0, pl.ds(0, 16)] = wv0
            for h in range(H):
                plsc.store_scatter(wbuf, [ei1, hvecs[h]], css1[h],
                                   mask=lane15)
            wv1 = jnp.exp(wbuf[e1, pl.ds(0, 16)])
            wbuf[e1, pl.ds(0, 16)] = wv1
            return 0
        lax.fori_loop(0, C // 2, _alpha, 0)

        if p == 0:
            @pl.when(j > 0)
            def _():
                pltpu.make_async_copy(vbn, accwv.at[dxn], sem_s[1 - p]).wait()
        else:
            pltpu.make_async_copy(vbn, accwv.at[dxn], sem_s[1 - p]).wait()
        nbase = base + (2 * j + p + 1) * C
        pltpu.async_copy(src_hbm.at[pl.ds(nbase, C)], sxn, sem_i1)
        pltpu.async_copy(dst_hbm.at[pl.ds(nbase, C)], dxn, sem_i2)
        pltpu.make_async_copy(src_hbm.at[pl.ds(nbase, C)], sxn, sem_i1).wait()
        pltpu.make_async_copy(dst_hbm.at[pl.ds(nbase, C)], dxn, sem_i2).wait()
        pltpu.async_copy(q_hbm.at[dxn], qb, sem_q)
        pltpu.async_copy(k_hbm.at[sxn], kb, sem_k)
        pltpu.async_copy(v_hbm.at[sxn], vbn, sem_v[1 - p])

        pltpu.make_async_copy(v_hbm.at[sx], vbp, sem_v[p]).wait()

        def _msg(i, _):
            for e in (2 * i, 2 * i + 1):
                wv = wbuf[e, pl.ds(0, 16)]
                whs = [_splat_idx(wv, hsplat[h]) for h in range(H)]
                vvs = [vbp[e, pl.ds(h * 16, 16)] for h in range(H)]
                for h in range(H):
                    vbp[e, pl.ds(h * 16, 16)] = vvs[h] * whs[h]
            return 0
        lax.fori_loop(0, C // 2, _msg, 0)

        pltpu.async_copy(vbp, accwv.at[dx], sem_s[p])

        def _stage(u, _):
            dvec = dx[pl.ds(u * 16, 16)]
            didx2[pl.ds(u * 16, 16)] = lax.shift_right_logical(dvec, 3)
            cbv = lax.shift_left(dvec & 7, 4)
            cbs = [cbv[l] for l in range(16)]
            wms = [wbuf[u * 16 + l, pl.ds(0, 16)] * mask8 for l in range(16)]
            for l in range(16):
                msgd[u * 16 + l, pl.ds(cbs[l], 16)] = wms[l]
            return 0
        lax.fori_loop(0, NG, _stage, 0)
        pltpu.sync_copy(msgd, accd.at[didx2], add=True)

        def _rez(u, _):
            dvec = dx[pl.ds(u * 16, 16)]
            cbv = lax.shift_left(dvec & 7, 4)
            cbs = [cbv[l] for l in range(16)]
            for l in range(16):
                msgd[u * 16 + l, pl.ds(cbs[l], 16)] = zero16
            return 0
        lax.fori_loop(0, NG, _rez, 0)

    def _dbl(j, _):
        _half_chunk(j, 0)
        _half_chunk(j, 1)
        return 0
    lax.fori_loop(0, NDBL, _dbl, 0)

    pltpu.make_async_copy(vb1, accwv.at[didx1], sem_s1).wait()
    pltpu.make_async_copy(q_hbm.at[didx0], qb, sem_q).wait()
    pltpu.make_async_copy(k_hbm.at[sidx0], kb, sem_k).wait()
    pltpu.make_async_copy(v_hbm.at[sidx0], vb0, sem_v0).wait()
    plsc.subcore_barrier()

    for t in range(14):
        m = sid * 14 + t

        @pl.when(m < NZW)
        def _():
            pltpu.sync_copy(accwv.at[pl.ds(m * C, C)], qb)
            pltpu.sync_copy(qb, outwv_hbm.at[cid, pl.ds(m * C, C)])

    pltpu.sync_copy(accd.at[pl.ds(sid * 80, C)], qb)
    pltpu.sync_copy(qb, outd_hbm.at[cid, pl.ds(sid * 80, C)])
    pltpu.sync_copy(accd.at[pl.ds(sid * 80 + C, 32)], qb.at[pl.ds(0, 32)])
    pltpu.sync_copy(qb.at[pl.ds(0, 32)],
                    outd_hbm.at[cid, pl.ds(sid * 80 + C, 32)])


def _stage2(q_tab, k_tab, v_tab, src, dst):
    mesh = plsc.VectorSubcoreMesh(core_axis_name="c", subcore_axis_name="s")
    kern = pl.kernel(
        _edge_body,
        out_type=[jax.ShapeDtypeStruct((NC, AW, D), jnp.float32),
                  jax.ShapeDtypeStruct((NC, AD, D), jnp.float32)],
        mesh=mesh,
        compiler_params=pltpu.CompilerParams(needs_layout_passes=False),
        scratch_types=[
            pltpu.VMEM((C,), jnp.int32),
            pltpu.VMEM((C,), jnp.int32),
            pltpu.VMEM((C,), jnp.int32),
            pltpu.VMEM((C,), jnp.int32),
            pltpu.VMEM((C,), jnp.int32),
            pltpu.VMEM((C, D), jnp.float32),
            pltpu.VMEM((C, D), jnp.float32),
            pltpu.VMEM((C, D), jnp.float32),
            pltpu.VMEM((C, D), jnp.float32),
            pltpu.VMEM((C, 16), jnp.float32),
            pltpu.VMEM((C, D), jnp.float32),
            pltpu.VMEM_SHARED((AW, D), jnp.float32),
            pltpu.VMEM_SHARED((AD, D), jnp.float32),
            pltpu.SemaphoreType.DMA,
            pltpu.SemaphoreType.DMA,
            pltpu.SemaphoreType.DMA,
            pltpu.SemaphoreType.DMA,
            pltpu.SemaphoreType.DMA,
            pltpu.SemaphoreType.DMA,
            pltpu.SemaphoreType.DMA,
            pltpu.SemaphoreType.DMA,
        ],
    )
    return kern(q_tab, k_tab, v_tab, src, dst)


def _post_body(d0, d1, wv0, wv1, xt_ref, rrep, wih, bih, whh, bhh, wo, bo,
               g3, b3n, w1, bm1, w2, bm2, out_ref):
    denom = d0[...] + d1[...]
    inv = 1.0 / (denom + 1e-16)
    inv_full = jnp.dot(inv, rrep[...], preferred_element_type=jnp.float32)
    agg = (wv0[...] + wv1[...]) * inv_full
    xt = xt_ref[...]
    z = (jnp.dot(agg, wih[...], preferred_element_type=jnp.float32) + bih[...]
         + jnp.dot(xt, whh[...], preferred_element_type=jnp.float32) + bhh[...])
    gate = 1.0 / (1.0 + jnp.exp(-z))
    upd = agg * gate
    mha = jnp.dot(upd, wo[...], preferred_element_type=jnp.float32) + bo[...]
    x_t = xt + mha
    h = _ln(x_t, g3[...], b3n[...])
    ff1 = jnp.maximum(
        jnp.dot(h, w1[...], preferred_element_type=jnp.float32) + bm1[...], 0.0)
    ff = jnp.dot(ff1, w2[...], preferred_element_type=jnp.float32) + bm2[...]
    out_ref[...] = x_t + ff


def _stage3(den0, den1, wv0, wv1, x_tgt, Wih, bih, Whh, bhh, Wo, bo, g3, b3n,
            W1, bm1, W2, bm2):
    rrep = (jnp.arange(H)[:, None] == (jnp.arange(D)[None, :] // DH)
            ).astype(jnp.float32)
    blk = pl.BlockSpec((BN, D), lambda i: (i, 0))
    blk8 = pl.BlockSpec((BN, H), lambda i: (i, 0))
    mat = pl.BlockSpec((D, D), lambda i: (0, 0))
    vec = pl.BlockSpec((D,), lambda i: (0,))
    return pl.pallas_call(
        _post_body,
        grid=(N // BN,),
        in_specs=[blk8, blk8, blk, blk, blk,
                  pl.BlockSpec((H, D), lambda i: (0, 0)),
                  mat, vec, mat, vec, mat, vec, vec, vec,
                  pl.BlockSpec((D, 4 * D), lambda i: (0, 0)),
                  pl.BlockSpec((4 * D,), lambda i: (0,)),
                  pl.BlockSpec((4 * D, D), lambda i: (0, 0)),
                  vec],
        out_specs=blk,
        out_shape=jax.ShapeDtypeStruct((N, D), jnp.float32),
    )(den0, den1, wv0, wv1, x_tgt, rrep,
      Wih, bih, Whh, bhh, Wo, bo, g3, b3n, W1, bm1, W2, bm2)


def kernel(x_src, x_tgt, edge_index, Wq, bq, Wk, bk, Wv, bv, Wih, bih,
           Whh, bhh, Wo, bo, g1, b1n, g3, b3n, W1, bm1, W2, bm2):
    q_tab, k_tab, v_tab = _stage1(x_src, x_tgt, Wq, bq, Wk, bk, Wv, bv,
                                  g1, b1n)
    zpad = jnp.zeros((NP - N, D), jnp.float32)
    q_tab = jnp.concatenate([q_tab, zpad])
    k_tab = jnp.concatenate([k_tab, zpad])
    v_tab = jnp.concatenate([v_tab, zpad])
    ipad = jnp.full((EPAD - E,), N, jnp.int32)
    src = jnp.concatenate([edge_index[0], ipad])
    dst = jnp.concatenate([edge_index[1], ipad])
    wv_parts, d_parts = _stage2(q_tab, k_tab, v_tab, src, dst)
    den = d_parts.reshape(NC, AD * 8, 16)[:, :N, :H]
    wv = wv_parts[:, :N, :]
    return _stage3(den[0], den[1], wv[0], wv[1], x_tgt,
                   Wih, bih, Whh, bhh, Wo, bo, g3, b3n, W1, bm1, W2, bm2)


# speedup vs baseline: 1.0422x; 1.0422x over previous
import jax
import jax.numpy as jnp
from jax import lax
from jax.experimental import pallas as pl
from jax.experimental.pallas import tpu as pltpu
from jax.experimental.pallas import tpu_sc as plsc

N = 10000
E = 320000
D = 128
H = 8
DH = D // H

NC = 2
NS = 16
NW = NC * NS
C = 48
NG = C // 16
NCHUNK = 210
NDBL = NCHUNK // 2
EPT = NCHUNK * C
EPAD = NW * EPT + C
NP = N + 16
AW = 10032
NZW = AW // C
AD = 1280
BN = 1000


_DNUMS = lax.GatherDimensionNumbers(
    offset_dims=(), collapsed_slice_dims=(0,), start_index_map=(0,))


def _splat_idx(vec, idx):
    return lax.gather(vec, idx, _DNUMS, (1,),
                      mode=lax.GatherScatterMode.PROMISE_IN_BOUNDS)


def _unpack8(ref, e, base):
    ws = [ref[e, pl.ds(base + j * 16, 16)] for j in range(4)]
    los = [plsc.bitcast(lax.shift_left(w, 16), jnp.float32) for w in ws]
    his = [plsc.bitcast(w & jnp.int32(-65536), jnp.float32) for w in ws]
    return los + his


def _ln(x, g, b):
    m = jnp.mean(x, axis=-1, keepdims=True)
    v = jnp.mean((x - m) * (x - m), axis=-1, keepdims=True)
    return (x - m) / jnp.sqrt(v + 1e-5) * g + b


def _pack_bf16(x):
    lo = x[:, :64].astype(jnp.bfloat16).astype(jnp.float32)
    hi = x[:, 64:].astype(jnp.bfloat16).astype(jnp.float32)
    lob = lax.bitcast_convert_type(lo, jnp.int32)
    hib = lax.bitcast_convert_type(hi, jnp.int32)
    return lax.shift_right_logical(lob, 16) | (hib & jnp.int32(-65536))


def _proj_body(xs_ref, xt_ref, wq, bq, wk, bk, wv, bv, g1, b1n,
               q_out, kv_out):
    xs = _ln(xs_ref[...], g1[...], b1n[...])
    mask = (jnp.sum(xs, axis=-1, keepdims=True) != 0).astype(jnp.float32)
    q_out[...] = jnp.dot(xt_ref[...], wq[...],
                         preferred_element_type=jnp.float32) + bq[...]
    k = (jnp.dot(xs, wk[...],
                 preferred_element_type=jnp.float32) + bk[...]) * 0.25
    v = (jnp.dot(xs, wv[...],
                 preferred_element_type=jnp.float32) + bv[...]) * mask
    kv_out[:, : D // 2] = _pack_bf16(k)
    kv_out[:, D // 2:] = _pack_bf16(v)


def _stage1(x_src, x_tgt, Wq, bq, Wk, bk, Wv, bv, g1, b1n):
    mat = pl.BlockSpec((D, D), lambda i: (0, 0))
    vec = pl.BlockSpec((D,), lambda i: (0,))
    blk = pl.BlockSpec((BN, D), lambda i: (i, 0))
    return pl.pallas_call(
        _proj_body,
        grid=(N // BN,),
        in_specs=[blk, blk, mat, vec, mat, vec, mat, vec, vec, vec],
        out_specs=[blk, blk],
        out_shape=[jax.ShapeDtypeStruct((N, D), jnp.float32),
                   jax.ShapeDtypeStruct((N, D), jnp.int32)],
    )(x_src, x_tgt, Wq, bq, Wk, bk, Wv, bv, g1, b1n)


def _edge_body(q_hbm, kv_hbm, src_hbm, dst_hbm, outwv_hbm, outd_hbm,
               sidx0, sidx1, didx0, didx1, didx2, qb, kvb0, kvb1,
               wvb0, wvb1, wbuf, msgd, accwv, accd,
               sem_q, sem_k, sem_i1, sem_i2, sem_s0, sem_s1):
    cid = lax.axis_index("c")
    sid = lax.axis_index("s")
    zero16 = jnp.zeros((16,), jnp.float32)
    lane = lax.iota(jnp.int32, 16)
    lane15 = lane == 15
    mask8 = (lane < H).astype(jnp.float32)
    hvecs = [jnp.full((16,), h, jnp.int32) for h in range(H)]
    hsplat = [jnp.full((16, 1), h, jnp.int32) for h in range(H)]

    sidx = (sidx0, sidx1)
    didx = (didx0, didx1)
    kvb = (kvb0, kvb1)
    wvb = (wvb0, wvb1)
    sem_s = (sem_s0, sem_s1)

    def _zrow(i, _):
        for j in range(D // 16):
            wvb0[i, pl.ds(j * 16, 16)] = zero16
        wbuf[i, pl.ds(0, 16)] = zero16
        return 0
    lax.fori_loop(0, C, _zrow, 0)

    def _zrow2(i, _):
        for j in range(D // 16):
            msgd[i, pl.ds(j * 16, 16)] = zero16
        return 0
    lax.fori_loop(0, 16, _zrow2, 0)

    for t in range(14):
        m = sid * 14 + t

        @pl.when(m < NZW)
        def _():
            pltpu.sync_copy(wvb0, accwv.at[pl.ds(m * C, C)])

    pltpu.sync_copy(wvb0, accd.at[pl.ds(sid * 80, C)])
    pltpu.sync_copy(wvb0.at[pl.ds(0, 32)], accd.at[pl.ds(sid * 80 + C, 32)])
    plsc.subcore_barrier()

    wid = sid * NC + cid
    base = wid * EPT

    pltpu.sync_copy(src_hbm.at[pl.ds(base, C)], sidx0)
    pltpu.sync_copy(dst_hbm.at[pl.ds(base, C)], didx0)
    pltpu.async_copy(q_hbm.at[didx0], qb, sem_q)
    pltpu.async_copy(kv_hbm.at[sidx0], kvb0, sem_k)

    def _half_chunk(j, p):
        sx, dx = sidx[p], didx[p]
        kvp, wvp = kvb[p], wvb[p]
        sxn, dxn = sidx[1 - p], didx[1 - p]

        pltpu.make_async_copy(q_hbm.at[dx], qb, sem_q).wait()
        pltpu.make_async_copy(kv_hbm.at[sx], kvp, sem_k).wait()

        def _alpha(i, _):
            e0 = 2 * i
            e1 = 2 * i + 1
            ei0 = lane * 0 + e0
            ei1 = lane * 0 + e1
            qs0 = [qb[e0, pl.ds(h * 16, 16)] for h in range(H)]
            ks0 = _unpack8(kvp, e0, 0)
            css0 = [plsc.cumsum(qs0[h] * ks0[h]) for h in range(H)]
            qs1 = [qb[e1, pl.ds(h * 16, 16)] for h in range(H)]
            ks1 = _unpack8(kvp, e1, 0)
            for h in range(H):
                plsc.store_scatter(wbuf, [ei0, hvecs[h]], css0[h],
                                   mask=lane15)
            css1 = [plsc.cumsum(qs1[h] * ks1[h]) for h in range(H)]
            wv0 = jnp.exp(wbuf[e0, pl.ds(0, 16)])
            wbuf[e0, pl.ds(0, 16)] = wv0
            for h in range(H):
                plsc.store_scatter(wbuf, [ei1, hvecs[h]], css1[h],
                                   mask=lane15)
            wv1 = jnp.exp(wbuf[e1, pl.ds(0, 16)])
            wbuf[e1, pl.ds(0, 16)] = wv1
            return 0
        lax.fori_loop(0, C // 2, _alpha, 0)

        nbase = base + (2 * j + p + 1) * C
        pltpu.async_copy(src_hbm.at[pl.ds(nbase, C)], sxn, sem_i1)
        pltpu.async_copy(dst_hbm.at[pl.ds(nbase, C)], dxn, sem_i2)
        pltpu.make_async_copy(src_hbm.at[pl.ds(nbase, C)], sxn, sem_i1).wait()
        pltpu.make_async_copy(dst_hbm.at[pl.ds(nbase, C)], dxn, sem_i2).wait()
        pltpu.async_copy(q_hbm.at[dxn], qb, sem_q)
        pltpu.async_copy(kv_hbm.at[sxn], kvb[1 - p], sem_k)

        @pl.when(j > 0)
        def _():
            pltpu.make_async_copy(wvp, accwv.at[dx], sem_s[p]).wait()

        def _msg(i, _):
            for e in (2 * i, 2 * i + 1):
                wv = wbuf[e, pl.ds(0, 16)]
                whs = [_splat_idx(wv, hsplat[h]) for h in range(H)]
                vvs = _unpack8(kvp, e, D // 2)
                for h in range(H):
                    wvp[e, pl.ds(h * 16, 16)] = vvs[h] * whs[h]
            return 0
        lax.fori_loop(0, C // 2, _msg, 0)

        pltpu.async_copy(wvp, accwv.at[dx], sem_s[p])

        for u in range(NG):
            dvec = dx[pl.ds(u * 16, 16)]
            didx2[pl.ds(0, 16)] = lax.shift_right_logical(dvec, 3)
            cbv = lax.shift_left(dvec & 7, 4)
            cbs = [cbv[l] for l in range(16)]
            wms = [wbuf[u * 16 + l, pl.ds(0, 16)] * mask8 for l in range(16)]
            for l in range(16):
                msgd[l, pl.ds(cbs[l], 16)] = wms[l]
            pltpu.sync_copy(msgd, accd.at[didx2], add=True)
            for l in range(16):
                msgd[l, pl.ds(cbs[l], 16)] = zero16

    def _dbl(j, _):
        _half_chunk(j, 0)
        _half_chunk(j, 1)
        return 0
    lax.fori_loop(0, NDBL, _dbl, 0)

    pltpu.make_async_copy(wvb0, accwv.at[didx0], sem_s0).wait()
    pltpu.make_async_copy(wvb1, accwv.at[didx1], sem_s1).wait()
    pltpu.make_async_copy(q_hbm.at[didx0], qb, sem_q).wait()
    pltpu.make_async_copy(kv_hbm.at[sidx0], kvb0, sem_k).wait()
    plsc.subcore_barrier()

    for t in range(14):
        m = sid * 14 + t

        @pl.when(m < NZW)
        def _():
            pltpu.sync_copy(accwv.at[pl.ds(m * C, C)], wvb0)
            pltpu.sync_copy(wvb0, outwv_hbm.at[cid, pl.ds(m * C, C)])

    pltpu.sync_copy(accd.at[pl.ds(sid * 80, C)], wvb1)
    pltpu.sync_copy(wvb1, outd_hbm.at[cid, pl.ds(sid * 80, C)])
    pltpu.sync_copy(accd.at[pl.ds(sid * 80 + C, 32)], wvb1.at[pl.ds(0, 32)])
    pltpu.sync_copy(wvb1.at[pl.ds(0, 32)],
                    outd_hbm.at[cid, pl.ds(sid * 80 + C, 32)])


def _stage2(q_tab, kv_tab, src, dst):
    mesh = plsc.VectorSubcoreMesh(core_axis_name="c", subcore_axis_name="s")
    kern = pl.kernel(
        _edge_body,
        out_type=[jax.ShapeDtypeStruct((NC, AW, D), jnp.float32),
                  jax.ShapeDtypeStruct((NC, AD, D), jnp.float32)],
        mesh=mesh,
        compiler_params=pltpu.CompilerParams(needs_layout_passes=False),
        scratch_types=[
            pltpu.VMEM((C,), jnp.int32),
            pltpu.VMEM((C,), jnp.int32),
            pltpu.VMEM((C,), jnp.int32),
            pltpu.VMEM((C,), jnp.int32),
            pltpu.VMEM((16,), jnp.int32),
            pltpu.VMEM((C, D), jnp.float32),
            pltpu.VMEM((C, D), jnp.int32),
            pltpu.VMEM((C, D), jnp.int32),
            pltpu.VMEM((C, D), jnp.float32),
            pltpu.VMEM((C, D), jnp.float32),
            pltpu.VMEM((C, 16), jnp.float32),
            pltpu.VMEM((16, D), jnp.float32),
            pltpu.VMEM_SHARED((AW, D), jnp.float32),
            pltpu.VMEM_SHARED((AD, D), jnp.float32),
            pltpu.SemaphoreType.DMA,
            pltpu.SemaphoreType.DMA,
            pltpu.SemaphoreType.DMA,
            pltpu.SemaphoreType.DMA,
            pltpu.SemaphoreType.DMA,
            pltpu.SemaphoreType.DMA,
        ],
    )
    return kern(q_tab, kv_tab, src, dst)


def _post_body(d0, d1, wv0, wv1, xt_ref, rrep, wih, bih, whh, bhh, wo, bo,
               g3, b3n, w1, bm1, w2, bm2, out_ref):
    denom = d0[...] + d1[...]
    inv = 1.0 / (denom + 1e-16)
    inv_full = jnp.dot(inv, rrep[...], preferred_element_type=jnp.float32)
    agg = (wv0[...] + wv1[...]) * inv_full
    xt = xt_ref[...]
    z = (jnp.dot(agg, wih[...], preferred_element_type=jnp.float32) + bih[...]
         + jnp.dot(xt, whh[...], preferred_element_type=jnp.float32) + bhh[...])
    gate = 1.0 / (1.0 + jnp.exp(-z))
    upd = agg * gate
    mha = jnp.dot(upd, wo[...], preferred_element_type=jnp.float32) + bo[...]
    x_t = xt + mha
    h = _ln(x_t, g3[...], b3n[...])
    ff1 = jnp.maximum(
        jnp.dot(h, w1[...], preferred_element_type=jnp.float32) + bm1[...], 0.0)
    ff = jnp.dot(ff1, w2[...], preferred_element_type=jnp.float32) + bm2[...]
    out_ref[...] = x_t + ff


def _stage3(den0, den1, wv0, wv1, x_tgt, Wih, bih, Whh, bhh, Wo, bo, g3, b3n,
            W1, bm1, W2, bm2):
    rrep = (jnp.arange(H)[:, None] == (jnp.arange(D)[None, :] // DH)
            ).astype(jnp.float32)
    blk = pl.BlockSpec((BN, D), lambda i: (i, 0))
    blk8 = pl.BlockSpec((BN, H), lambda i: (i, 0))
    mat = pl.BlockSpec((D, D), lambda i: (0, 0))
    vec = pl.BlockSpec((D,), lambda i: (0,))
    return pl.pallas_call(
        _post_body,
        grid=(N // BN,),
        in_specs=[blk8, blk8, blk, blk, blk,
                  pl.BlockSpec((H, D), lambda i: (0, 0)),
                  mat, vec, mat, vec, mat, vec, vec, vec,
                  pl.BlockSpec((D, 4 * D), lambda i: (0, 0)),
                  pl.BlockSpec((4 * D,), lambda i: (0,)),
                  pl.BlockSpec((4 * D, D), lambda i: (0, 0)),
                  vec],
        out_specs=blk,
        out_shape=jax.ShapeDtypeStruct((N, D), jnp.float32),
    )(den0, den1, wv0, wv1, x_tgt, rrep,
      Wih, bih, Whh, bhh, Wo, bo, g3, b3n, W1, bm1, W2, bm2)


def kernel(x_src, x_tgt, edge_index, Wq, bq, Wk, bk, Wv, bv, Wih, bih,
           Whh, bhh, Wo, bo, g1, b1n, g3, b3n, W1, bm1, W2, bm2):
    q_tab, kv_tab = _stage1(x_src, x_tgt, Wq, bq, Wk, bk, Wv, bv, g1, b1n)
    q_tab = jnp.concatenate([q_tab, jnp.zeros((NP - N, D), jnp.float32)])
    kv_tab = jnp.concatenate([kv_tab, jnp.zeros((NP - N, D), jnp.int32)])
    ipad = jnp.full((EPAD - E,), N, jnp.int32)
    src = jnp.concatenate([edge_index[0], ipad])
    dst = jnp.concatenate([edge_index[1], ipad])
    wv_parts, d_parts = _stage2(q_tab, kv_tab, src, dst)
    den = d_parts.reshape(NC, AD * 8, 16)[:, :N, :H]
    wv = wv_parts[:, :N, :]
    return _stage3(den[0], den[1], wv[0], wv[1], x_tgt,
                   Wih, bih, Whh, bhh, Wo, bo, g3, b3n, W1, bm1, W2, bm2)
